# baseline (speedup 1.0000x reference)
import functools

import jax
import jax.numpy as jnp
from jax.experimental import pallas as pl
from jax.experimental.pallas import tpu as pltpu

N = 10000
E = 320000
H = 8


def _proj_kernel(x_ref, wl_ref, wr_ref, xl_ref, xr_ref):
    x = x_ref[...]
    xl_ref[...] = jnp.dot(x, wl_ref[...], preferred_element_type=jnp.float32)
    xr_ref[...] = jnp.dot(x, wr_ref[...], preferred_element_type=jnp.float32)


def _project(x, Wl, Wr):
    n, fin = x.shape
    fout = Wl.shape[1]
    bn = 1000
    grid = (n // bn,)
    return pl.pallas_call(
        _proj_kernel,
        grid=grid,
        in_specs=[
            pl.BlockSpec((bn, fin), lambda i: (i, 0)),
            pl.BlockSpec((fin, fout), lambda i: (0, 0)),
            pl.BlockSpec((fin, fout), lambda i: (0, 0)),
        ],
        out_specs=[
            pl.BlockSpec((bn, fout), lambda i: (i, 0)),
            pl.BlockSpec((bn, fout), lambda i: (i, 0)),
        ],
        out_shape=[
            jax.ShapeDtypeStruct((n, fout), jnp.float32),
            jax.ShapeDtypeStruct((n, fout), jnp.float32),
        ],
    )(x, Wl, Wr)


def _edge_layer(xl, xr, src, dst, att, b, heads, fout, n):
    e = jax.nn.leaky_relu(xl[src] + xr[dst], negative_slope=0.2)
    score = jnp.einsum('ehf,hf->eh', e.reshape(-1, heads, fout), att)
    smax = jax.ops.segment_max(score, dst, num_segments=n)
    ex = jnp.exp(score - smax[dst])
    den = jax.ops.segment_sum(ex, dst, num_segments=n)
    alpha = ex / (den[dst] + 1e-16)
    msg = xl[src].reshape(-1, heads, fout) * alpha[:, :, None]
    out = jax.ops.segment_sum(msg, dst, num_segments=n)
    return out.reshape(n, heads * fout) + b


def kernel(x, edge_index, Wl1, Wr1, att1, b1, Wl2, Wr2, att2, b2):
    loop = jnp.arange(N, dtype=edge_index.dtype)
    src = jnp.concatenate([edge_index[0], loop])
    dst = jnp.concatenate([edge_index[1], loop])

    xl1, xr1 = _project(x, Wl1, Wr1)
    h = _edge_layer(xl1, xr1, src, dst, att1, b1, H, 128, N)
    h = jax.nn.elu(h)
    xl2, xr2 = _project(h, Wl2, Wr2)
    h = _edge_layer(xl2, xr2, src, dst, att2, b2, H, 128, N)
    return jax.nn.log_softmax(h, axis=1)


# speedup vs baseline: 5.4842x; 5.4842x over previous
import functools

import jax
import jax.numpy as jnp
from jax import lax
from jax.experimental import pallas as pl
from jax.experimental.pallas import tpu as pltpu
from jax.experimental.pallas import tpu_sc as plsc

N = 10000
E = 320000
ET = E + N
H = 8
F = 128
D = H * F

NC = 2
NS = 16
NW = NC * NS
NPW = 320
RPBUF = 336
RPPAD = 31 * NPW + RPBUF


def _proj_kernel(x_ref, wl_ref, wr_ref, xl_ref, xr_ref):
    x = x_ref[...]
    xl_ref[...] = jnp.dot(x, wl_ref[...], preferred_element_type=jnp.float32)
    xr_ref[...] = jnp.dot(x, wr_ref[...], preferred_element_type=jnp.float32)


def _proj2_kernel(h_ref, b_ref, wl_ref, wr_ref, xl_ref, xr_ref):
    z = h_ref[...] + b_ref[...]
    t = jnp.where(z > 0, z, jnp.exp(jnp.minimum(z, 0.0)) - 1.0)
    xl_ref[...] = jnp.dot(t, wl_ref[...], preferred_element_type=jnp.float32)
    xr_ref[...] = jnp.dot(t, wr_ref[...], preferred_element_type=jnp.float32)


def _project(x, Wl, Wr, b=None):
    n, fin = x.shape
    fout = Wl.shape[1]
    bn = 1000
    grid = (n // bn,)
    if b is None:
        body = _proj_kernel
        args = (x, Wl, Wr)
        in_specs = [
            pl.BlockSpec((bn, fin), lambda i: (i, 0)),
            pl.BlockSpec((fin, fout), lambda i: (0, 0)),
            pl.BlockSpec((fin, fout), lambda i: (0, 0)),
        ]
    else:
        body = _proj2_kernel
        args = (x, b, Wl, Wr)
        in_specs = [
            pl.BlockSpec((bn, fin), lambda i: (i, 0)),
            pl.BlockSpec((1, fin), lambda i: (0, 0)),
            pl.BlockSpec((fin, fout), lambda i: (0, 0)),
            pl.BlockSpec((fin, fout), lambda i: (0, 0)),
        ]
    return pl.pallas_call(
        body,
        grid=grid,
        in_specs=in_specs,
        out_specs=[
            pl.BlockSpec((bn, fout), lambda i: (i, 0)),
            pl.BlockSpec((bn, fout), lambda i: (i, 0)),
        ],
        out_shape=[
            jax.ShapeDtypeStruct((n, fout), jnp.float32),
            jax.ShapeDtypeStruct((n, fout), jnp.float32),
        ],
    )(*args)


def _lsm_kernel(h_ref, b_ref, o_ref):
    z = h_ref[...] + b_ref[...]
    m = jnp.max(z, axis=1, keepdims=True)
    zc = z - m
    lse = jnp.log(jnp.sum(jnp.exp(zc), axis=1, keepdims=True))
    o_ref[...] = zc - lse


def _log_softmax(h, b):
    n, d = h.shape
    bn = 1000
    return pl.pallas_call(
        _lsm_kernel,
        grid=(n // bn,),
        in_specs=[
            pl.BlockSpec((bn, d), lambda i: (i, 0)),
            pl.BlockSpec((1, d), lambda i: (0, 0)),
        ],
        out_specs=pl.BlockSpec((bn, d), lambda i: (i, 0)),
        out_shape=jax.ShapeDtypeStruct((n, d), jnp.float32),
    )(h, b)


def _shuf(v, perm):
    dn = lax.GatherDimensionNumbers(offset_dims=(), collapsed_slice_dims=(0,),
                                    start_index_map=(0,))
    return lax.gather(v, perm[:, None], dn, (1,),
                      mode=lax.GatherScatterMode.PROMISE_IN_BOUNDS)


def _sc_body(xl_h, xr_h, att_h, src_h, rp_h, out_h,
             rp_v, idx_v, xlb, xrv, attv, accv, ext_v):
    cid = lax.axis_index("c")
    sid = lax.axis_index("s")
    wid = sid * NC + cid
    d_lo = wid * NPW
    d_hi = jnp.minimum(d_lo + NPW, N)
    pltpu.sync_copy(rp_h.at[pl.ds(d_lo, RPBUF)], rp_v)
    pltpu.sync_copy(att_h, attv)
    lanes = lax.iota(jnp.int32, 16)
    zlanes = lanes * 0
    zero16 = jnp.full((16,), 0.0, jnp.float32)

    def _allsum(v):
        for k in (8, 4, 2, 1):
            v = v + _shuf(v, lanes ^ k)
        return v

    def _extract(i):
        base = (i // 16) * 16
        v = rp_v[pl.ds(base, 16)]
        ext_v[...] = _shuf(v, zlanes + (i - base))
        return ext_v[...][0]

    def node_body(d, loaded_in):
        li = d - d_lo
        e0 = _extract(li)
        e1 = _extract(li + 1)
        pltpu.sync_copy(xr_h.at[d], xrv)
        for h in range(H):
            for j in range(F // 16):
                accv[h, pl.ds(j * 16, 16)] = zero16

        def edge_body(e, carry):
            m0s, den, loaded = carry
            chunk = e // 16
            pos = e - chunk * 16

            @pl.when(chunk != loaded)
            def _():
                pltpu.sync_copy(src_h.at[pl.ds(chunk * 16, 16)], idx_v)
                pltpu.sync_copy(xl_h.at[idx_v], xlb)

            m0s_new = []
            den_new = den
            for h in range(H):
                p = zero16
                for j in range(F // 16):
                    sl = pl.ds(h * F + j * 16, 16)
                    z = xlb[pos, sl] + xrv[sl]
                    l = jnp.maximum(z, z * 0.2)
                    p = p + l * attv[h, pl.ds(j * 16, 16)]
                sv = _allsum(p)
                m0v = jnp.where(e == e0, sv, m0s[h])
                m0s_new.append(m0v)
                wb = jnp.exp(sv - m0v)
                den_new = jnp.where(lanes == h, den_new + wb, den_new)
                for j in range(F // 16):
                    sl = pl.ds(h * F + j * 16, 16)
                    plsc.addupdate(accv.at[h, pl.ds(j * 16, 16)],
                                   wb * xlb[pos, sl])
            return (tuple(m0s_new), den_new, chunk)

        init = (tuple(zero16 for _ in range(H)), zero16, loaded_in)
        _, den, loaded_out = lax.fori_loop(e0, e1, edge_body, init)

        for h in range(H):
            dh = _shuf(den, zlanes + h)
            inv = 1.0 / (dh + 1e-16)
            for j in range(F // 16):
                sl = pl.ds(j * 16, 16)
                accv[h, sl] = accv[h, sl] * inv
        pltpu.sync_copy(accv, out_h.at[d])
        return loaded_out

    lax.fori_loop(d_lo, d_hi, node_body, jnp.int32(-1))


@functools.partial(jax.jit)
def _sc_edge_layer(xl, xr, att, src_s, rowptr_pad):
    mesh = plsc.VectorSubcoreMesh(core_axis_name="c", subcore_axis_name="s")
    return pl.kernel(
        _sc_body,
        out_type=jax.ShapeDtypeStruct((N, H, F), jnp.float32),
        mesh=mesh,
        scratch_types=[
            pltpu.VMEM((RPBUF,), jnp.int32),
            pltpu.VMEM((16,), jnp.int32),
            pltpu.VMEM((16, D), jnp.float32),
            pltpu.VMEM((D,), jnp.float32),
            pltpu.VMEM((H, F), jnp.float32),
            pltpu.VMEM((H, F), jnp.float32),
            pltpu.VMEM((16,), jnp.int32),
        ],
    )(xl, xr, att, src_s, rowptr_pad)


def kernel(x, edge_index, Wl1, Wr1, att1, b1, Wl2, Wr2, att2, b2):
    loop = jnp.arange(N, dtype=edge_index.dtype)
    src = jnp.concatenate([edge_index[0], loop]).astype(jnp.int32)
    dst = jnp.concatenate([edge_index[1], loop]).astype(jnp.int32)
    dst_s, src_s = lax.sort((dst, src), num_keys=1)
    rowptr = jnp.searchsorted(
        dst_s, jnp.arange(N + 1, dtype=jnp.int32)).astype(jnp.int32)
    rowptr_pad = jnp.concatenate(
        [rowptr, jnp.full((RPPAD - (N + 1),), ET, jnp.int32)])

    xl1, xr1 = _project(x, Wl1, Wr1)
    o1 = _sc_edge_layer(xl1, xr1, att1, src_s, rowptr_pad)
    h1 = o1.reshape(N, D)
    xl2, xr2 = _project(h1, Wl2, Wr2, b=b1.reshape(1, D))
    o2 = _sc_edge_layer(xl2, xr2, att2, src_s, rowptr_pad)
    h2 = o2.reshape(N, D)
    return _log_softmax(h2, b2.reshape(1, D))


# speedup vs baseline: 7.3155x; 1.3339x over previous
import functools

import jax
import jax.numpy as jnp
from jax import lax
from jax.experimental import pallas as pl
from jax.experimental.pallas import tpu as pltpu
from jax.experimental.pallas import tpu_sc as plsc

N = 10000
E = 320000
ET = E + N
H = 8
F = 128
D = H * F

NC = 2
NS = 16
NW = NC * NS
NPW = 320
RPBUF = 336
RPPAD = 31 * NPW + RPBUF


def _proj_kernel(x_ref, wl_ref, wr_ref, xl_ref, xr_ref):
    x = x_ref[...]
    xl_ref[...] = jnp.dot(x, wl_ref[...], preferred_element_type=jnp.float32)
    xr_ref[...] = jnp.dot(x, wr_ref[...], preferred_element_type=jnp.float32)


def _proj2_kernel(h_ref, b_ref, wl_ref, wr_ref, xl_ref, xr_ref):
    z = h_ref[...] + b_ref[...]
    t = jnp.where(z > 0, z, jnp.exp(jnp.minimum(z, 0.0)) - 1.0)
    xl_ref[...] = jnp.dot(t, wl_ref[...], preferred_element_type=jnp.float32)
    xr_ref[...] = jnp.dot(t, wr_ref[...], preferred_element_type=jnp.float32)


def _project(x, Wl, Wr, b=None):
    n, fin = x.shape
    fout = Wl.shape[1]
    bn = 1000
    grid = (n // bn,)
    if b is None:
        body = _proj_kernel
        args = (x, Wl, Wr)
        in_specs = [
            pl.BlockSpec((bn, fin), lambda i: (i, 0)),
            pl.BlockSpec((fin, fout), lambda i: (0, 0)),
            pl.BlockSpec((fin, fout), lambda i: (0, 0)),
        ]
    else:
        body = _proj2_kernel
        args = (x, b, Wl, Wr)
        in_specs = [
            pl.BlockSpec((bn, fin), lambda i: (i, 0)),
            pl.BlockSpec((1, fin), lambda i: (0, 0)),
            pl.BlockSpec((fin, fout), lambda i: (0, 0)),
            pl.BlockSpec((fin, fout), lambda i: (0, 0)),
        ]
    return pl.pallas_call(
        body,
        grid=grid,
        in_specs=in_specs,
        out_specs=[
            pl.BlockSpec((bn, fout), lambda i: (i, 0)),
            pl.BlockSpec((bn, fout), lambda i: (i, 0)),
        ],
        out_shape=[
            jax.ShapeDtypeStruct((n, fout), jnp.float32),
            jax.ShapeDtypeStruct((n, fout), jnp.float32),
        ],
    )(*args)


def _lsm_kernel(h_ref, b_ref, o_ref):
    z = h_ref[...] + b_ref[...]
    m = jnp.max(z, axis=1, keepdims=True)
    zc = z - m
    lse = jnp.log(jnp.sum(jnp.exp(zc), axis=1, keepdims=True))
    o_ref[...] = zc - lse


def _log_softmax(h, b):
    n, d = h.shape
    bn = 1000
    return pl.pallas_call(
        _lsm_kernel,
        grid=(n // bn,),
        in_specs=[
            pl.BlockSpec((bn, d), lambda i: (i, 0)),
            pl.BlockSpec((1, d), lambda i: (0, 0)),
        ],
        out_specs=pl.BlockSpec((bn, d), lambda i: (i, 0)),
        out_shape=jax.ShapeDtypeStruct((n, d), jnp.float32),
    )(h, b)


def _shuf(v, perm):
    dn = lax.GatherDimensionNumbers(offset_dims=(), collapsed_slice_dims=(0,),
                                    start_index_map=(0,))
    return lax.gather(v, perm[:, None], dn, (1,),
                      mode=lax.GatherScatterMode.PROMISE_IN_BOUNDS)


def _sc_body(xl_h, xr_h, att_h, src_h, rp_h, out_h,
             rp_v, idx2, xlb2, xrv, attv, accv, ext_v, sem):
    cid = lax.axis_index("c")
    sid = lax.axis_index("s")
    wid = sid * NC + cid
    d_lo = wid * NPW
    d_hi = jnp.minimum(d_lo + NPW, N)
    pltpu.sync_copy(rp_h.at[pl.ds(d_lo, RPBUF)], rp_v)
    pltpu.sync_copy(att_h, attv)
    lanes = lax.iota(jnp.int32, 16)
    zlanes = lanes * 0
    zero16 = jnp.full((16,), 0.0, jnp.float32)

    def _allsum(v):
        for k in (8, 4, 2, 1):
            v = v + _shuf(v, lanes ^ k)
        return v

    def _extract(i):
        base = (i // 16) * 16
        v = rp_v[pl.ds(base, 16)]
        ext_v[...] = _shuf(v, zlanes + (i - base))
        return ext_v[...][0]

    def _issue(c):
        pltpu.sync_copy(src_h.at[pl.ds(c * 16, 16)], idx2.at[c % 2])
        pltpu.async_copy(xl_h.at[idx2.at[c % 2]], xlb2.at[c % 2],
                         sem.at[c % 2])

    def _wait(c):
        pltpu.make_async_copy(xl_h.at[idx2.at[c % 2]], xlb2.at[c % 2],
                              sem.at[c % 2]).wait()

    c0 = _extract(0) // 16
    _issue(c0)

    def node_body(d, loaded_in):
        li = d - d_lo
        e0 = _extract(li)
        e1 = _extract(li + 1)
        pltpu.sync_copy(xr_h.at[d], xrv)
        for h in range(H):
            for j in range(F // 16):
                accv[h, pl.ds(j * 16, 16)] = zero16

        def edge_body(e, carry):
            m0v, den, loaded = carry
            chunk = e // 16
            pos = e - chunk * 16
            slot = chunk % 2

            @pl.when(chunk != loaded)
            def _():
                _wait(chunk)
                _issue(chunk + 1)

            svec = zero16
            for h in range(H):
                p = zero16
                for j in range(F // 16):
                    sl = pl.ds(h * F + j * 16, 16)
                    z = xlb2[slot, pos, sl] + xrv[sl]
                    l = jnp.maximum(z, z * 0.2)
                    p = p + l * attv[h, pl.ds(j * 16, 16)]
                sv = _allsum(p)
                svec = jnp.where(lanes == h, sv, svec)
            m0v = jnp.where(e == e0, svec, m0v)
            wv = jnp.exp(svec - m0v)
            den = den + wv
            for h in range(H):
                wb = _shuf(wv, zlanes + h)
                for j in range(F // 16):
                    sl = pl.ds(h * F + j * 16, 16)
                    plsc.addupdate(accv.at[h, pl.ds(j * 16, 16)],
                                   wb * xlb2[slot, pos, sl])
            return (m0v, den, chunk)

        init = (zero16, zero16, loaded_in)
        _, den, loaded_out = lax.fori_loop(e0, e1, edge_body, init)

        for h in range(H):
            dh = _shuf(den, zlanes + h)
            inv = 1.0 / (dh + 1e-16)
            for j in range(F // 16):
                sl = pl.ds(j * 16, 16)
                accv[h, sl] = accv[h, sl] * inv
        pltpu.sync_copy(accv, out_h.at[d])
        return loaded_out

    last = lax.fori_loop(d_lo, d_hi, node_body, jnp.int32(-1))
    _wait(last + 1)


@functools.partial(jax.jit)
def _sc_edge_layer(xl, xr, att, src_s, rowptr_pad):
    mesh = plsc.VectorSubcoreMesh(core_axis_name="c", subcore_axis_name="s")
    return pl.kernel(
        _sc_body,
        out_type=jax.ShapeDtypeStruct((N, H, F), jnp.float32),
        mesh=mesh,
        scratch_types=[
            pltpu.VMEM((RPBUF,), jnp.int32),
            pltpu.VMEM((2, 16), jnp.int32),
            pltpu.VMEM((2, 16, D), jnp.float32),
            pltpu.VMEM((D,), jnp.float32),
            pltpu.VMEM((H, F), jnp.float32),
            pltpu.VMEM((H, F), jnp.float32),
            pltpu.VMEM((16,), jnp.int32),
            pltpu.SemaphoreType.DMA((2,)),
        ],
    )(xl, xr, att, src_s, rowptr_pad)


def kernel(x, edge_index, Wl1, Wr1, att1, b1, Wl2, Wr2, att2, b2):
    loop = jnp.arange(N, dtype=edge_index.dtype)
    src = jnp.concatenate([edge_index[0], loop]).astype(jnp.int32)
    dst = jnp.concatenate([edge_index[1], loop]).astype(jnp.int32)
    dst_s, src_s = lax.sort((dst, src), num_keys=1)
    src_s = jnp.concatenate([src_s, jnp.zeros((16,), jnp.int32)])
    rowptr = jnp.searchsorted(
        dst_s, jnp.arange(N + 1, dtype=jnp.int32)).astype(jnp.int32)
    rowptr_pad = jnp.concatenate(
        [rowptr, jnp.full((RPPAD - (N + 1),), ET, jnp.int32)])

    xl1, xr1 = _project(x, Wl1, Wr1)
    o1 = _sc_edge_layer(xl1, xr1, att1, src_s, rowptr_pad)
    h1 = o1.reshape(N, D)
    xl2, xr2 = _project(h1, Wl2, Wr2, b=b1.reshape(1, D))
    o2 = _sc_edge_layer(xl2, xr2, att2, src_s, rowptr_pad)
    h2 = o2.reshape(N, D)
    return _log_softmax(h2, b2.reshape(1, D))
